# baseline (speedup 1.0000x reference)
import functools

import jax
import jax.numpy as jnp
from jax import lax
from jax.experimental import pallas as pl
from jax.experimental.pallas import tpu as pltpu
from jax.experimental.pallas import tpu_sc as plsc

N = 10000
E = 320000
D = 128
R = 8
EPS = 1e-5

NC = 2
NS = 16
K_PER_SC = 5
CHUNK = 1024
ROWS = CHUNK * R
ROWSP = ROWS + 16
SROWS = NC * K_PER_SC * ROWS
EPT = E // NS
EBLK = 800
NEB = EPT // EBLK
B = 128
RING = 16
RPT = ROWS // NS


def _sc_body(with_gather, *args):
    if with_gather:
        (x_hbm, src_hbm, dst_hbm, typ_hbm, s_hbm, acc_sh, dstv, typv, srcv,
         sel_src, sel_row, rows_v0, rows_v1, gsem0, gsem1) = args
    else:
        (dst_hbm, typ_hbm, s_hbm, acc_sh, dstv, typv, sel_row,
         rows_v0) = args
    cid = lax.axis_index("c")
    sid = lax.axis_index("s")
    iota16 = lax.iota(jnp.int32, 16)
    zero16 = jnp.zeros((16,), jnp.float32)
    one16 = jnp.ones((16,), jnp.float32)
    RMASK = RING * B - 1

    def fill_rows(val, nrows):
        def fr(j, carry):
            for cc in range(D // 16):
                rows_v0[j, pl.ds(cc * 16, 16)] = val
            return carry
        lax.fori_loop(0, nrows, fr, 0)

    ebase = sid * EPT
    db = sid * RPT

    def sel_row_at(k):
        return sel_row.at[k & (RING - 1)]

    if with_gather:
        def sel_src_at(k):
            return sel_src.at[k & (RING - 1)]

        def gath(k):
            @pl.when((k & 1) == 0)
            def _():
                pltpu.async_copy(x_hbm.at[sel_src_at(k)], rows_v0, gsem0)
            @pl.when((k & 1) == 1)
            def _():
                pltpu.async_copy(x_hbm.at[sel_src_at(k)], rows_v1, gsem1)

        def wgath(k):
            @pl.when((k & 1) == 0)
            def _():
                pltpu.make_async_copy(x_hbm.at[sel_src_at(k)], rows_v0,
                                      gsem0).wait()
            @pl.when((k & 1) == 1)
            def _():
                pltpu.make_async_copy(x_hbm.at[sel_src_at(k)], rows_v1,
                                      gsem1).wait()

        def scat_sync(k):
            @pl.when((k & 1) == 0)
            def _():
                pltpu.sync_copy(rows_v0, acc_sh.at[sel_row_at(k)], add=True)
            @pl.when((k & 1) == 1)
            def _():
                pltpu.sync_copy(rows_v1, acc_sh.at[sel_row_at(k)], add=True)

        def flush(k0, k1):
            @pl.when(k1 > k0)
            def _():
                gath(k0)

                def body(k, carry):
                    wgath(k)
                    @pl.when(k + 1 < k1)
                    def _():
                        gath(k + 1)
                    scat_sync(k)
                    return carry
                lax.fori_loop(k0, k1, body, 0)
            return k1
    else:
        def flush(k0, k1):
            def body(k, carry):
                pltpu.sync_copy(rows_v0, acc_sh.at[sel_row_at(k)], add=True)
                return carry
            lax.fori_loop(k0, k1, body, 0)
            return k1

    def chunk_body(ci, carry):
        c = cid * K_PER_SC + ci
        lo = c * CHUNK

        fill_rows(zero16, 64)
        for t in range(RPT // 64):
            pltpu.sync_copy(rows_v0.at[pl.ds(0, 64)],
                            acc_sh.at[pl.ds(db + t * 64, 64)])
        if not with_gather:
            fill_rows(one16, B)
        plsc.subcore_barrier()

        def block_body(bi, carry):
            cnt, kdone = carry
            eoff = ebase + bi * EBLK
            pltpu.sync_copy(dst_hbm.at[pl.ds(eoff, EBLK)], dstv)
            pltpu.sync_copy(typ_hbm.at[pl.ds(eoff, EBLK)], typv)
            if with_gather:
                pltpu.sync_copy(src_hbm.at[pl.ds(eoff, EBLK)], srcv)

            def scan_body(i, cnt):
                d = dstv[pl.ds(i * 16, 16)]
                t = typv[pl.ds(i * 16, 16)]
                dl = d - lo
                m = (dl >= 0) & (dl < CHUNK)
                mi = m.astype(jnp.int32)
                pre = plsc.cumsum(mi) - mi
                rp = (cnt + pre) & RMASK
                row = dl * R + t
                if with_gather:
                    plsc.store_scatter(sel_src, [rp >> 7, rp & (B - 1)],
                                       srcv[pl.ds(i * 16, 16)], mask=m)
                plsc.store_scatter(sel_row, [rp >> 7, rp & (B - 1)], row,
                                   mask=m)
                return cnt + jnp.sum(mi)
            cnt = lax.fori_loop(0, EBLK // 16, scan_body, cnt)
            kdone = flush(kdone, cnt >> 7)
            return (cnt, kdone)
        cnt, kdone = lax.fori_loop(0, NEB, block_body,
                                   (jnp.int32(0), jnp.int32(0)))

        nb = (cnt + (B - 1)) // B
        pad = nb * B - cnt
        for j in range(B // 16):
            p = (cnt + j * 16 + iota16) & RMASK
            mpad = (j * 16 + iota16) < pad
            if with_gather:
                plsc.store_scatter(sel_src, [p >> 7, p & (B - 1)],
                                   sid * 16 + iota16, mask=mpad)
            plsc.store_scatter(sel_row, [p >> 7, p & (B - 1)],
                               ROWS + iota16, mask=mpad)
        flush(kdone, nb)
        plsc.subcore_barrier()

        hb = c * ROWS + db
        pltpu.sync_copy(acc_sh.at[pl.ds(db, RPT)],
                        s_hbm.at[pl.ds(hb, RPT)])
        return carry
    lax.fori_loop(0, K_PER_SC, chunk_body, 0)


def _make_sc_kernel(with_gather):
    mesh = plsc.VectorSubcoreMesh(core_axis_name="c", subcore_axis_name="s",
                                  num_cores=NC, num_subcores=NS)
    if with_gather:
        scratch = (
            pltpu.VMEM_SHARED((ROWSP, D), jnp.float32),
            pltpu.VMEM((EBLK,), jnp.int32),
            pltpu.VMEM((EBLK,), jnp.int32),
            pltpu.VMEM((EBLK,), jnp.int32),
            pltpu.VMEM((RING, B), jnp.int32),
            pltpu.VMEM((RING, B), jnp.int32),
            pltpu.VMEM((B, D), jnp.float32),
            pltpu.VMEM((B, D), jnp.float32),
            pltpu.SemaphoreType.DMA,
            pltpu.SemaphoreType.DMA,
        )
    else:
        scratch = (
            pltpu.VMEM_SHARED((ROWSP, D), jnp.float32),
            pltpu.VMEM((EBLK,), jnp.int32),
            pltpu.VMEM((EBLK,), jnp.int32),
            pltpu.VMEM((RING, B), jnp.int32),
            pltpu.VMEM((B, D), jnp.float32),
        )
    return pl.kernel(
        functools.partial(_sc_body, with_gather),
        out_type=(jax.ShapeDtypeStruct((SROWS, D), jnp.float32),),
        mesh=mesh,
        compiler_params=pltpu.CompilerParams(needs_layout_passes=False),
        scratch_types=scratch,
    )


_sc_layer = _make_sc_kernel(True)
_sc_cnt = _make_sc_kernel(False)

BN = 1000


def _tc_body(leaky, x_ref, s_ref, c_ref, root_ref, w_ref, b_ref, g_ref,
             bt_ref, o_ref):
    x = x_ref[...]
    acc = jnp.dot(x, root_ref[...], preferred_element_type=jnp.float32)
    acc = acc + b_ref[...]
    rec = jnp.concatenate(
        [jnp.broadcast_to(jnp.maximum(c_ref[:, r * D:r * D + 1], 1.0),
                          (BN, D)) for r in range(R)], axis=1)
    mean = s_ref[...] / rec
    acc = acc + jnp.dot(mean, w_ref[...], preferred_element_type=jnp.float32)
    if leaky:
        acc = jnp.where(acc > 0, acc, 0.2 * acc)
    mu = jnp.mean(acc, axis=1, keepdims=True)
    var = jnp.mean((acc - mu) ** 2, axis=1, keepdims=True)
    o_ref[...] = (acc - mu) / jnp.sqrt(var + EPS) * g_ref[...] + bt_ref[...]


def _make_tc_kernel(leaky):
    return pl.pallas_call(
        functools.partial(_tc_body, leaky),
        grid=(N // BN,),
        in_specs=[
            pl.BlockSpec((BN, D), lambda i: (i, 0)),
            pl.BlockSpec((BN, R * D), lambda i: (i, 0)),
            pl.BlockSpec((BN, R * D), lambda i: (i, 0)),
            pl.BlockSpec((D, D), lambda i: (0, 0)),
            pl.BlockSpec((R * D, D), lambda i: (0, 0)),
            pl.BlockSpec((1, D), lambda i: (0, 0)),
            pl.BlockSpec((1, D), lambda i: (0, 0)),
            pl.BlockSpec((1, D), lambda i: (0, 0)),
        ],
        out_specs=pl.BlockSpec((BN, D), lambda i: (i, 0)),
        out_shape=jax.ShapeDtypeStruct((N, D), jnp.float32),
    )


_tc_layer1 = _make_tc_kernel(True)
_tc_layer2 = _make_tc_kernel(False)


def kernel(x, edge_index, edge_type, W1, root1, b1, g1, beta1, W2, root2,
           b2, g2, beta2):
    src = edge_index[0]
    dst = edge_index[1]
    (cnt,) = _sc_cnt(dst, edge_type)
    cs = cnt[:N * R].reshape(N, R * D)
    (s1,) = _sc_layer(x, src, dst, edge_type)
    h = _tc_layer1(x, s1[:N * R].reshape(N, R * D), cs, root1,
                   W1.reshape(R * D, D), b1.reshape(1, D),
                   g1.reshape(1, D), beta1.reshape(1, D))
    (s2,) = _sc_layer(h, src, dst, edge_type)
    out = _tc_layer2(h, s2[:N * R].reshape(N, R * D), cs, root2,
                     W2.reshape(R * D, D), b2.reshape(1, D),
                     g2.reshape(1, D), beta2.reshape(1, D))
    return out


# speedup vs baseline: 1.6140x; 1.6140x over previous
import functools

import jax
import jax.numpy as jnp
from jax import lax
from jax.experimental import pallas as pl
from jax.experimental.pallas import tpu as pltpu
from jax.experimental.pallas import tpu_sc as plsc

N = 10000
E = 320000
D = 128
R = 8
EPS = 1e-5

NC = 2
NS = 16
K_PER_SC = 5
CHUNK = 1024
ROWS = CHUNK * R
ROWSP = ROWS + 16
SROWS = NC * K_PER_SC * ROWS
EPT = E // NS
EBLK = 800
NEB = EPT // EBLK
B = 128
RING = 16
RPT = ROWS // NS


def _sc_body(with_gather, *args):
    if with_gather:
        (x_hbm, key_hbm, src_hbm, s_hbm, acc_sh, keyv0, keyv1, srcv0,
         srcv1, sel_src, sel_row, rows_v0, rows_v1, gsem0, gsem1, esem0,
         esem1) = args
    else:
        (key_hbm, s_hbm, acc_sh, keyv0, keyv1, sel_row, rows_v0, esem0,
         esem1) = args
    cid = lax.axis_index("c")
    sid = lax.axis_index("s")
    iota16 = lax.iota(jnp.int32, 16)
    zero16 = jnp.zeros((16,), jnp.float32)
    one16 = jnp.ones((16,), jnp.float32)
    RMASK = RING * B - 1

    def fill_rows(val, nrows):
        def fr(j, carry):
            for cc in range(D // 16):
                rows_v0[j, pl.ds(cc * 16, 16)] = val
            return carry
        lax.fori_loop(0, nrows, fr, 0)

    ebase = sid * EPT
    db = sid * RPT

    def sel_row_at(k):
        return sel_row.at[k & (RING - 1)]

    if with_gather:
        def sel_src_at(k):
            return sel_src.at[k & (RING - 1)]

        def gath(k):
            @pl.when((k & 1) == 0)
            def _():
                pltpu.async_copy(x_hbm.at[sel_src_at(k)], rows_v0, gsem0)
            @pl.when((k & 1) == 1)
            def _():
                pltpu.async_copy(x_hbm.at[sel_src_at(k)], rows_v1, gsem1)

        def wgath(k):
            @pl.when((k & 1) == 0)
            def _():
                pltpu.make_async_copy(x_hbm.at[sel_src_at(k)], rows_v0,
                                      gsem0).wait()
            @pl.when((k & 1) == 1)
            def _():
                pltpu.make_async_copy(x_hbm.at[sel_src_at(k)], rows_v1,
                                      gsem1).wait()

        def scat_sync(k):
            @pl.when((k & 1) == 0)
            def _():
                pltpu.sync_copy(rows_v0, acc_sh.at[sel_row_at(k)], add=True)
            @pl.when((k & 1) == 1)
            def _():
                pltpu.sync_copy(rows_v1, acc_sh.at[sel_row_at(k)], add=True)

        def flush(k0, k1):
            @pl.when(k1 > k0)
            def _():
                gath(k0)

                def body(k, carry):
                    wgath(k)
                    @pl.when(k + 1 < k1)
                    def _():
                        gath(k + 1)
                    scat_sync(k)
                    return carry
                lax.fori_loop(k0, k1, body, 0)
            return k1
    else:
        def flush(k0, k1):
            def body(k, carry):
                pltpu.sync_copy(rows_v0, acc_sh.at[sel_row_at(k)], add=True)
                return carry
            lax.fori_loop(k0, k1, body, 0)
            return k1

    def chunk_body(ci, carry):
        c = cid * K_PER_SC + ci
        lo = c * CHUNK

        fill_rows(zero16, 64)
        for t in range(RPT // 64):
            pltpu.sync_copy(rows_v0.at[pl.ds(0, 64)],
                            acc_sh.at[pl.ds(db + t * 64, 64)])
        if not with_gather:
            fill_rows(one16, B)
        plsc.subcore_barrier()

        lo8 = lo * R

        def stage(bi):
            eoff = ebase + bi * EBLK
            @pl.when((bi & 1) == 0)
            def _():
                pltpu.async_copy(key_hbm.at[pl.ds(eoff, EBLK)], keyv0,
                                 esem0)
                if with_gather:
                    pltpu.async_copy(src_hbm.at[pl.ds(eoff, EBLK)], srcv0,
                                     esem0)
            @pl.when((bi & 1) == 1)
            def _():
                pltpu.async_copy(key_hbm.at[pl.ds(eoff, EBLK)], keyv1,
                                 esem1)
                if with_gather:
                    pltpu.async_copy(src_hbm.at[pl.ds(eoff, EBLK)], srcv1,
                                     esem1)

        def wstage(bi):
            eoff = ebase + bi * EBLK
            @pl.when((bi & 1) == 0)
            def _():
                pltpu.make_async_copy(key_hbm.at[pl.ds(eoff, EBLK)], keyv0,
                                      esem0).wait()
                if with_gather:
                    pltpu.make_async_copy(src_hbm.at[pl.ds(eoff, EBLK)],
                                          srcv0, esem0).wait()
            @pl.when((bi & 1) == 1)
            def _():
                pltpu.make_async_copy(key_hbm.at[pl.ds(eoff, EBLK)], keyv1,
                                      esem1).wait()
                if with_gather:
                    pltpu.make_async_copy(src_hbm.at[pl.ds(eoff, EBLK)],
                                          srcv1, esem1).wait()

        def scan_block(keyv, srcv, cnt):
            def scan_body(i, cnt):
                kl = keyv[pl.ds(i * 16, 16)] - lo8
                m = (kl >= 0) & (kl < ROWS)
                mi = m.astype(jnp.int32)
                pre = plsc.cumsum(mi) - mi
                rp = (cnt + pre) & RMASK
                if with_gather:
                    plsc.store_scatter(sel_src, [rp >> 7, rp & (B - 1)],
                                       srcv[pl.ds(i * 16, 16)], mask=m)
                plsc.store_scatter(sel_row, [rp >> 7, rp & (B - 1)], kl,
                                   mask=m)
                return cnt + jnp.sum(mi)
            return lax.fori_loop(0, EBLK // 16, scan_body, cnt)

        stage(0)

        def block_body(bi, carry):
            cnt, kdone = carry
            wstage(bi)
            @pl.when(bi + 1 < NEB)
            def _():
                stage(bi + 1)
            cnt = lax.cond((bi & 1) == 0,
                           lambda c: scan_block(keyv0, srcv0
                                                if with_gather else None, c),
                           lambda c: scan_block(keyv1, srcv1
                                                if with_gather else None, c),
                           cnt)
            kfull = cnt >> 7
            k1 = jnp.where(kfull - kdone >= 8, kfull, kdone)
            kdone = flush(kdone, k1)
            return (cnt, kdone)
        cnt, kdone = lax.fori_loop(0, NEB, block_body,
                                   (jnp.int32(0), jnp.int32(0)))

        nb = (cnt + (B - 1)) // B
        pad = nb * B - cnt
        for j in range(B // 16):
            p = (cnt + j * 16 + iota16) & RMASK
            mpad = (j * 16 + iota16) < pad
            if with_gather:
                plsc.store_scatter(sel_src, [p >> 7, p & (B - 1)],
                                   sid * 16 + iota16, mask=mpad)
            plsc.store_scatter(sel_row, [p >> 7, p & (B - 1)],
                               ROWS + iota16, mask=mpad)
        flush(kdone, nb)
        plsc.subcore_barrier()

        hb = c * ROWS + db
        pltpu.sync_copy(acc_sh.at[pl.ds(db, RPT)],
                        s_hbm.at[pl.ds(hb, RPT)])
        return carry
    lax.fori_loop(0, K_PER_SC, chunk_body, 0)


def _make_sc_kernel(with_gather):
    mesh = plsc.VectorSubcoreMesh(core_axis_name="c", subcore_axis_name="s",
                                  num_cores=NC, num_subcores=NS)
    if with_gather:
        scratch = (
            pltpu.VMEM_SHARED((ROWSP, D), jnp.float32),
            pltpu.VMEM((EBLK,), jnp.int32),
            pltpu.VMEM((EBLK,), jnp.int32),
            pltpu.VMEM((EBLK,), jnp.int32),
            pltpu.VMEM((EBLK,), jnp.int32),
            pltpu.VMEM((RING, B), jnp.int32),
            pltpu.VMEM((RING, B), jnp.int32),
            pltpu.VMEM((B, D), jnp.float32),
            pltpu.VMEM((B, D), jnp.float32),
            pltpu.SemaphoreType.DMA,
            pltpu.SemaphoreType.DMA,
            pltpu.SemaphoreType.DMA,
            pltpu.SemaphoreType.DMA,
        )
    else:
        scratch = (
            pltpu.VMEM_SHARED((ROWSP, D), jnp.float32),
            pltpu.VMEM((EBLK,), jnp.int32),
            pltpu.VMEM((EBLK,), jnp.int32),
            pltpu.VMEM((RING, B), jnp.int32),
            pltpu.VMEM((B, D), jnp.float32),
            pltpu.SemaphoreType.DMA,
            pltpu.SemaphoreType.DMA,
        )
    return pl.kernel(
        functools.partial(_sc_body, with_gather),
        out_type=(jax.ShapeDtypeStruct((SROWS, D), jnp.float32),),
        mesh=mesh,
        compiler_params=pltpu.CompilerParams(needs_layout_passes=False),
        scratch_types=scratch,
    )


_sc_layer = _make_sc_kernel(True)
_sc_cnt = _make_sc_kernel(False)

BN = 1000


def _tc_body(leaky, x_ref, s_ref, c_ref, root_ref, w_ref, b_ref, g_ref,
             bt_ref, o_ref):
    x = x_ref[...]
    acc = jnp.dot(x, root_ref[...], preferred_element_type=jnp.float32)
    acc = acc + b_ref[...]
    rec = jnp.concatenate(
        [jnp.broadcast_to(jnp.maximum(c_ref[:, r * D:r * D + 1], 1.0),
                          (BN, D)) for r in range(R)], axis=1)
    mean = s_ref[...] / rec
    acc = acc + jnp.dot(mean, w_ref[...], preferred_element_type=jnp.float32)
    if leaky:
        acc = jnp.where(acc > 0, acc, 0.2 * acc)
    mu = jnp.mean(acc, axis=1, keepdims=True)
    var = jnp.mean((acc - mu) ** 2, axis=1, keepdims=True)
    o_ref[...] = (acc - mu) / jnp.sqrt(var + EPS) * g_ref[...] + bt_ref[...]


def _make_tc_kernel(leaky):
    return pl.pallas_call(
        functools.partial(_tc_body, leaky),
        grid=(N // BN,),
        in_specs=[
            pl.BlockSpec((BN, D), lambda i: (i, 0)),
            pl.BlockSpec((BN, R * D), lambda i: (i, 0)),
            pl.BlockSpec((BN, R * D), lambda i: (i, 0)),
            pl.BlockSpec((D, D), lambda i: (0, 0)),
            pl.BlockSpec((R * D, D), lambda i: (0, 0)),
            pl.BlockSpec((1, D), lambda i: (0, 0)),
            pl.BlockSpec((1, D), lambda i: (0, 0)),
            pl.BlockSpec((1, D), lambda i: (0, 0)),
        ],
        out_specs=pl.BlockSpec((BN, D), lambda i: (i, 0)),
        out_shape=jax.ShapeDtypeStruct((N, D), jnp.float32),
    )


_tc_layer1 = _make_tc_kernel(True)
_tc_layer2 = _make_tc_kernel(False)


def kernel(x, edge_index, edge_type, W1, root1, b1, g1, beta1, W2, root2,
           b2, g2, beta2):
    src = edge_index[0]
    dst = edge_index[1]
    key = dst * R + edge_type
    (cnt,) = _sc_cnt(key)
    cs = cnt[:N * R].reshape(N, R * D)
    (s1,) = _sc_layer(x, key, src)
    h = _tc_layer1(x, s1[:N * R].reshape(N, R * D), cs, root1,
                   W1.reshape(R * D, D), b1.reshape(1, D),
                   g1.reshape(1, D), beta1.reshape(1, D))
    (s2,) = _sc_layer(h, key, src)
    out = _tc_layer2(h, s2[:N * R].reshape(N, R * D), cs, root2,
                     W2.reshape(R * D, D), b2.reshape(1, D),
                     g2.reshape(1, D), beta2.reshape(1, D))
    return out


# speedup vs baseline: 1.7372x; 1.0763x over previous
import functools

import jax
import jax.numpy as jnp
from jax import lax
from jax.experimental import pallas as pl
from jax.experimental.pallas import tpu as pltpu
from jax.experimental.pallas import tpu_sc as plsc

N = 10000
E = 320000
D = 128
R = 8
EPS = 1e-5

NC = 2
NS = 16
K_PER_SC = 5
CHUNK = 1024
ROWS = CHUNK * R
ROWSP = ROWS + 16
SROWS = NC * K_PER_SC * ROWS
EPT = E // NS
EBLK = 800
NEB = EPT // EBLK
B = 128
RING = 16
RPT = ROWS // NS


def _sc_body(with_gather, *args):
    if with_gather:
        (x_hbm, key_hbm, src_hbm, s_hbm, acc_sh, keyv0, keyv1, srcv0,
         srcv1, sel_src, sel_row, rows_v0, rows_v1, gsem0, gsem1, esem0,
         esem1) = args
    else:
        (key_hbm, s_hbm, acc_sh, keyv0, keyv1, sel_row, rows_v0, esem0,
         esem1) = args
    cid = lax.axis_index("c")
    sid = lax.axis_index("s")
    iota16 = lax.iota(jnp.int32, 16)
    zero16 = jnp.zeros((16,), jnp.float32)
    one16 = jnp.ones((16,), jnp.float32)
    RMASK = RING * B - 1

    def fill_rows(val, nrows):
        def fr(j, carry):
            for cc in range(D // 16):
                rows_v0[j, pl.ds(cc * 16, 16)] = val
            return carry
        lax.fori_loop(0, nrows, fr, 0)

    ebase = sid * EPT
    db = sid * RPT

    def sel_row_at(k):
        return sel_row.at[k & (RING - 1)]

    if with_gather:
        def sel_src_at(k):
            return sel_src.at[k & (RING - 1)]

        def gath(k):
            @pl.when((k & 1) == 0)
            def _():
                pltpu.async_copy(x_hbm.at[sel_src_at(k)], rows_v0, gsem0)
            @pl.when((k & 1) == 1)
            def _():
                pltpu.async_copy(x_hbm.at[sel_src_at(k)], rows_v1, gsem1)

        def wgath(k):
            @pl.when((k & 1) == 0)
            def _():
                pltpu.make_async_copy(x_hbm.at[sel_src_at(k)], rows_v0,
                                      gsem0).wait()
            @pl.when((k & 1) == 1)
            def _():
                pltpu.make_async_copy(x_hbm.at[sel_src_at(k)], rows_v1,
                                      gsem1).wait()

        def scat_sync(k):
            @pl.when((k & 1) == 0)
            def _():
                pltpu.sync_copy(rows_v0, acc_sh.at[sel_row_at(k)], add=True)
            @pl.when((k & 1) == 1)
            def _():
                pltpu.sync_copy(rows_v1, acc_sh.at[sel_row_at(k)], add=True)

        def flush(k0, k1):
            @pl.when(k1 > k0)
            def _():
                gath(k0)

                def body(k, carry):
                    wgath(k)
                    @pl.when(k + 1 < k1)
                    def _():
                        gath(k + 1)
                    scat_sync(k)
                    return carry
                lax.fori_loop(k0, k1, body, 0)
            return k1
    else:
        def flush(k0, k1):
            def body(k, carry):
                pltpu.sync_copy(rows_v0, acc_sh.at[sel_row_at(k)], add=True)
                return carry
            lax.fori_loop(k0, k1, body, 0)
            return k1

    def chunk_body(ci, carry):
        c = cid * K_PER_SC + ci
        lo = c * CHUNK

        fill_rows(zero16, 64)
        for t in range(RPT // 64):
            pltpu.sync_copy(rows_v0.at[pl.ds(0, 64)],
                            acc_sh.at[pl.ds(db + t * 64, 64)])
        if not with_gather:
            fill_rows(one16, B)
        plsc.subcore_barrier()

        lo8 = lo * R

        def stage(bi):
            eoff = ebase + bi * EBLK
            @pl.when((bi & 1) == 0)
            def _():
                pltpu.async_copy(key_hbm.at[pl.ds(eoff, EBLK)], keyv0,
                                 esem0)
                if with_gather:
                    pltpu.async_copy(src_hbm.at[pl.ds(eoff, EBLK)], srcv0,
                                     esem0)
            @pl.when((bi & 1) == 1)
            def _():
                pltpu.async_copy(key_hbm.at[pl.ds(eoff, EBLK)], keyv1,
                                 esem1)
                if with_gather:
                    pltpu.async_copy(src_hbm.at[pl.ds(eoff, EBLK)], srcv1,
                                     esem1)

        def wstage(bi):
            eoff = ebase + bi * EBLK
            @pl.when((bi & 1) == 0)
            def _():
                pltpu.make_async_copy(key_hbm.at[pl.ds(eoff, EBLK)], keyv0,
                                      esem0).wait()
                if with_gather:
                    pltpu.make_async_copy(src_hbm.at[pl.ds(eoff, EBLK)],
                                          srcv0, esem0).wait()
            @pl.when((bi & 1) == 1)
            def _():
                pltpu.make_async_copy(key_hbm.at[pl.ds(eoff, EBLK)], keyv1,
                                      esem1).wait()
                if with_gather:
                    pltpu.make_async_copy(src_hbm.at[pl.ds(eoff, EBLK)],
                                          srcv1, esem1).wait()

        def scan_block(keyv, srcv, cnt):
            def scan_body(i, cnt):
                kl = keyv[pl.ds(i * 16, 16)] - lo8
                m = (kl >= 0) & (kl < ROWS)
                mi = m.astype(jnp.int32)
                pre = plsc.cumsum(mi) - mi
                rp = (cnt + pre) & RMASK
                if with_gather:
                    plsc.store_scatter(sel_src, [rp >> 7, rp & (B - 1)],
                                       srcv[pl.ds(i * 16, 16)], mask=m)
                plsc.store_scatter(sel_row, [rp >> 7, rp & (B - 1)], kl,
                                   mask=m)
                return cnt + jnp.sum(mi)
            return lax.fori_loop(0, EBLK // 16, scan_body, cnt)

        stage(0)

        def block_body(bi, carry):
            cnt, kdone = carry
            wstage(bi)
            @pl.when(bi + 1 < NEB)
            def _():
                stage(bi + 1)
            cnt = lax.cond((bi & 1) == 0,
                           lambda c: scan_block(keyv0, srcv0
                                                if with_gather else None, c),
                           lambda c: scan_block(keyv1, srcv1
                                                if with_gather else None, c),
                           cnt)
            kfull = cnt >> 7
            k1 = jnp.where(kfull - kdone >= 8, kfull, kdone)
            kdone = flush(kdone, k1)
            return (cnt, kdone)
        cnt, kdone = lax.fori_loop(0, NEB, block_body,
                                   (jnp.int32(0), jnp.int32(0)))

        nb = (cnt + (B - 1)) // B
        pad = nb * B - cnt
        for j in range(B // 16):
            p = (cnt + j * 16 + iota16) & RMASK
            mpad = (j * 16 + iota16) < pad
            if with_gather:
                plsc.store_scatter(sel_src, [p >> 7, p & (B - 1)],
                                   sid * 16 + iota16, mask=mpad)
            plsc.store_scatter(sel_row, [p >> 7, p & (B - 1)],
                               ROWS + iota16, mask=mpad)
        flush(kdone, nb)
        plsc.subcore_barrier()

        hb = c * ROWS + db
        pltpu.sync_copy(acc_sh.at[pl.ds(db, RPT)],
                        s_hbm.at[pl.ds(hb, RPT)])
        return carry
    lax.fori_loop(0, K_PER_SC, chunk_body, 0)


def _make_sc_kernel(with_gather):
    mesh = plsc.VectorSubcoreMesh(core_axis_name="c", subcore_axis_name="s",
                                  num_cores=NC, num_subcores=NS)
    if with_gather:
        scratch = (
            pltpu.VMEM_SHARED((ROWSP, D), jnp.float32),
            pltpu.VMEM((EBLK,), jnp.int32),
            pltpu.VMEM((EBLK,), jnp.int32),
            pltpu.VMEM((EBLK,), jnp.int32),
            pltpu.VMEM((EBLK,), jnp.int32),
            pltpu.VMEM((RING, B), jnp.int32),
            pltpu.VMEM((RING, B), jnp.int32),
            pltpu.VMEM((B, D), jnp.float32),
            pltpu.VMEM((B, D), jnp.float32),
            pltpu.SemaphoreType.DMA,
            pltpu.SemaphoreType.DMA,
            pltpu.SemaphoreType.DMA,
            pltpu.SemaphoreType.DMA,
        )
    else:
        scratch = (
            pltpu.VMEM_SHARED((ROWSP, D), jnp.float32),
            pltpu.VMEM((EBLK,), jnp.int32),
            pltpu.VMEM((EBLK,), jnp.int32),
            pltpu.VMEM((RING, B), jnp.int32),
            pltpu.VMEM((B, D), jnp.float32),
            pltpu.SemaphoreType.DMA,
            pltpu.SemaphoreType.DMA,
        )
    return pl.kernel(
        functools.partial(_sc_body, with_gather),
        out_type=(jax.ShapeDtypeStruct((SROWS, D), jnp.float32),),
        mesh=mesh,
        compiler_params=pltpu.CompilerParams(needs_layout_passes=False),
        scratch_types=scratch,
    )


_sc_layer = _make_sc_kernel(True)
_sc_cnt = _make_sc_kernel(False)

BN = 1000


def _tc_body(leaky, x_ref, s_ref, c_ref, root_ref, w_ref, b_ref, g_ref,
             bt_ref, o_ref):
    x = x_ref[...]
    acc = jnp.dot(x, root_ref[...], preferred_element_type=jnp.float32)
    acc = acc + b_ref[...]
    rec = jnp.concatenate(
        [jnp.broadcast_to(jnp.maximum(c_ref[:, r * D:r * D + 1], 1.0),
                          (BN, D)) for r in range(R)], axis=1)
    mean = s_ref[...] / rec
    acc = acc + jnp.dot(mean, w_ref[...], preferred_element_type=jnp.float32)
    if leaky:
        acc = jnp.where(acc > 0, acc, 0.2 * acc)
    mu = jnp.mean(acc, axis=1, keepdims=True)
    var = jnp.mean((acc - mu) ** 2, axis=1, keepdims=True)
    o_ref[...] = (acc - mu) / jnp.sqrt(var + EPS) * g_ref[...] + bt_ref[...]


def _make_tc_kernel(leaky):
    return pl.pallas_call(
        functools.partial(_tc_body, leaky),
        grid=(N // BN,),
        in_specs=[
            pl.BlockSpec((BN, D), lambda i: (i, 0)),
            pl.BlockSpec((BN, R * D), lambda i: (i, 0)),
            pl.BlockSpec((BN, R * D), lambda i: (i, 0)),
            pl.BlockSpec((D, D), lambda i: (0, 0)),
            pl.BlockSpec((R * D, D), lambda i: (0, 0)),
            pl.BlockSpec((1, D), lambda i: (0, 0)),
            pl.BlockSpec((1, D), lambda i: (0, 0)),
            pl.BlockSpec((1, D), lambda i: (0, 0)),
        ],
        out_specs=pl.BlockSpec((BN, D), lambda i: (i, 0)),
        out_shape=jax.ShapeDtypeStruct((N, D), jnp.float32),
    )


_tc_layer1 = _make_tc_kernel(True)
_tc_layer2 = _make_tc_kernel(False)


def kernel(x, edge_index, edge_type, W1, root1, b1, g1, beta1, W2, root2,
           b2, g2, beta2):
    src = edge_index[0]
    dst = edge_index[1]
    key = dst * R + edge_type
    NP = SROWS // R
    (cnt,) = _sc_cnt(key)
    cs = cnt.reshape(NP, R * D)
    (s1,) = _sc_layer(x, key, src)
    h = _tc_layer1(x, s1.reshape(NP, R * D), cs, root1,
                   W1.reshape(R * D, D), b1.reshape(1, D),
                   g1.reshape(1, D), beta1.reshape(1, D))
    (s2,) = _sc_layer(h, key, src)
    out = _tc_layer2(h, s2.reshape(NP, R * D), cs, root2,
                     W2.reshape(R * D, D), b2.reshape(1, D),
                     g2.reshape(1, D), beta2.reshape(1, D))
    return out


# speedup vs baseline: 1.8005x; 1.0365x over previous
import functools

import jax
import jax.numpy as jnp
from jax import lax
from jax.experimental import pallas as pl
from jax.experimental.pallas import tpu as pltpu
from jax.experimental.pallas import tpu_sc as plsc

N = 10000
E = 320000
D = 128
R = 8
EPS = 1e-5

NC = 2
NS = 16
K_PER_SC = 5
CHUNK = 1024
ROWS = CHUNK * R
ROWSP = ROWS + 16
SROWS = NC * K_PER_SC * ROWS
EPT = E // NS
EBLK = 800
NEB = EPT // EBLK
B = 128
RING = 16
RPT = ROWS // NS


def _sc_body(with_gather, *args):
    if with_gather:
        (x_hbm, key_hbm, src_hbm, s_hbm, acc_sh, keyv0, keyv1, srcv0,
         srcv1, sel_src, sel_row, rows_v0, rows_v1, gsem0, gsem1, esem0,
         esem1, ssem0, ssem1) = args
    else:
        (key_hbm, s_hbm, acc_sh, keyv0, keyv1, sel_row, rows_v0, esem0,
         esem1, ssem0) = args
    cid = lax.axis_index("c")
    sid = lax.axis_index("s")
    iota16 = lax.iota(jnp.int32, 16)
    zero16 = jnp.zeros((16,), jnp.float32)
    one16 = jnp.ones((16,), jnp.float32)
    RMASK = RING * B - 1

    def fill_rows(val, nrows):
        def fr(j, carry):
            for cc in range(D // 16):
                rows_v0[j, pl.ds(cc * 16, 16)] = val
            return carry
        lax.fori_loop(0, nrows, fr, 0)

    ebase = sid * EPT
    db = sid * RPT

    def sel_row_at(k):
        return sel_row.at[k & (RING - 1)]

    if with_gather:
        def sel_src_at(k):
            return sel_src.at[k & (RING - 1)]

        def gath(k):
            @pl.when((k & 1) == 0)
            def _():
                pltpu.async_copy(x_hbm.at[sel_src_at(k)], rows_v0, gsem0)
            @pl.when((k & 1) == 1)
            def _():
                pltpu.async_copy(x_hbm.at[sel_src_at(k)], rows_v1, gsem1)

        def wgath(k):
            @pl.when((k & 1) == 0)
            def _():
                pltpu.make_async_copy(x_hbm.at[sel_src_at(k)], rows_v0,
                                      gsem0).wait()
            @pl.when((k & 1) == 1)
            def _():
                pltpu.make_async_copy(x_hbm.at[sel_src_at(k)], rows_v1,
                                      gsem1).wait()

        def scat(k):
            @pl.when((k & 1) == 0)
            def _():
                pltpu.async_copy(rows_v0, acc_sh.at[sel_row_at(k)], ssem0,
                                 add=True)
            @pl.when((k & 1) == 1)
            def _():
                pltpu.async_copy(rows_v1, acc_sh.at[sel_row_at(k)], ssem1,
                                 add=True)

        def wscat(k):
            @pl.when((k & 1) == 0)
            def _():
                pltpu.make_async_copy(rows_v0, acc_sh.at[sel_row_at(k)],
                                      ssem0).wait()
            @pl.when((k & 1) == 1)
            def _():
                pltpu.make_async_copy(rows_v1, acc_sh.at[sel_row_at(k)],
                                      ssem1).wait()

        def flush(k0, k1):
            @pl.when(k1 > k0)
            def _():
                gath(k0)

                def body(k, carry):
                    @pl.when(k + 1 < k1)
                    def _():
                        @pl.when(k > k0)
                        def _():
                            wscat(k - 1)
                        gath(k + 1)
                    wgath(k)
                    scat(k)
                    return carry
                lax.fori_loop(k0, k1, body, 0)
                wscat(k1 - 1)
                @pl.when(k1 - 1 > k0)
                def _():
                    wscat(k1 - 2)
            return k1
    else:
        DEPTH = 4

        def one_wait(k):
            pltpu.make_async_copy(rows_v0, acc_sh.at[sel_row_at(k)],
                                  ssem0).wait()

        def flush(k0, k1):
            def body(k, carry):
                pltpu.async_copy(rows_v0, acc_sh.at[sel_row_at(k)], ssem0,
                                 add=True)
                @pl.when(k - k0 >= DEPTH)
                def _():
                    one_wait(k - DEPTH)
                return carry
            lax.fori_loop(k0, k1, body, 0)
            n = jnp.minimum(k1 - k0, DEPTH)
            for j in range(DEPTH):
                @pl.when(j < n)
                def _():
                    one_wait(k1 - n + j)
            return k1

    def chunk_body(ci, carry):
        c = cid * K_PER_SC + ci
        lo = c * CHUNK

        fill_rows(zero16, 64)
        for t in range(RPT // 64):
            pltpu.sync_copy(rows_v0.at[pl.ds(0, 64)],
                            acc_sh.at[pl.ds(db + t * 64, 64)])
        if not with_gather:
            fill_rows(one16, B)
        plsc.subcore_barrier()

        lo8 = lo * R

        def stage(bi):
            eoff = ebase + bi * EBLK
            @pl.when((bi & 1) == 0)
            def _():
                pltpu.async_copy(key_hbm.at[pl.ds(eoff, EBLK)], keyv0,
                                 esem0)
                if with_gather:
                    pltpu.async_copy(src_hbm.at[pl.ds(eoff, EBLK)], srcv0,
                                     esem0)
            @pl.when((bi & 1) == 1)
            def _():
                pltpu.async_copy(key_hbm.at[pl.ds(eoff, EBLK)], keyv1,
                                 esem1)
                if with_gather:
                    pltpu.async_copy(src_hbm.at[pl.ds(eoff, EBLK)], srcv1,
                                     esem1)

        def wstage(bi):
            eoff = ebase + bi * EBLK
            @pl.when((bi & 1) == 0)
            def _():
                pltpu.make_async_copy(key_hbm.at[pl.ds(eoff, EBLK)], keyv0,
                                      esem0).wait()
                if with_gather:
                    pltpu.make_async_copy(src_hbm.at[pl.ds(eoff, EBLK)],
                                          srcv0, esem0).wait()
            @pl.when((bi & 1) == 1)
            def _():
                pltpu.make_async_copy(key_hbm.at[pl.ds(eoff, EBLK)], keyv1,
                                      esem1).wait()
                if with_gather:
                    pltpu.make_async_copy(src_hbm.at[pl.ds(eoff, EBLK)],
                                          srcv1, esem1).wait()

        def scan_block(keyv, srcv, cnt):
            def scan_body(i, cnt):
                kl = keyv[pl.ds(i * 16, 16)] - lo8
                m = (kl >= 0) & (kl < ROWS)
                mi = m.astype(jnp.int32)
                pre = plsc.cumsum(mi) - mi
                rp = (cnt + pre) & RMASK
                if with_gather:
                    plsc.store_scatter(sel_src, [rp >> 7, rp & (B - 1)],
                                       srcv[pl.ds(i * 16, 16)], mask=m)
                plsc.store_scatter(sel_row, [rp >> 7, rp & (B - 1)], kl,
                                   mask=m)
                return cnt + jnp.sum(mi)
            return lax.fori_loop(0, EBLK // 16, scan_body, cnt)

        stage(0)

        def block_body(bi, carry):
            cnt, kdone = carry
            wstage(bi)
            @pl.when(bi + 1 < NEB)
            def _():
                stage(bi + 1)
            cnt = lax.cond((bi & 1) == 0,
                           lambda c: scan_block(keyv0, srcv0
                                                if with_gather else None, c),
                           lambda c: scan_block(keyv1, srcv1
                                                if with_gather else None, c),
                           cnt)
            kfull = cnt >> 7
            k1 = jnp.where(kfull - kdone >= 8, kfull, kdone)
            kdone = flush(kdone, k1)
            return (cnt, kdone)
        cnt, kdone = lax.fori_loop(0, NEB, block_body,
                                   (jnp.int32(0), jnp.int32(0)))

        nb = (cnt + (B - 1)) // B
        pad = nb * B - cnt
        for j in range(B // 16):
            p = (cnt + j * 16 + iota16) & RMASK
            mpad = (j * 16 + iota16) < pad
            if with_gather:
                plsc.store_scatter(sel_src, [p >> 7, p & (B - 1)],
                                   sid * 16 + iota16, mask=mpad)
            plsc.store_scatter(sel_row, [p >> 7, p & (B - 1)],
                               ROWS + iota16, mask=mpad)
        flush(kdone, nb)
        plsc.subcore_barrier()

        hb = c * ROWS + db
        pltpu.sync_copy(acc_sh.at[pl.ds(db, RPT)],
                        s_hbm.at[pl.ds(hb, RPT)])
        return carry
    lax.fori_loop(0, K_PER_SC, chunk_body, 0)


def _make_sc_kernel(with_gather):
    mesh = plsc.VectorSubcoreMesh(core_axis_name="c", subcore_axis_name="s",
                                  num_cores=NC, num_subcores=NS)
    if with_gather:
        scratch = (
            pltpu.VMEM_SHARED((ROWSP, D), jnp.float32),
            pltpu.VMEM((EBLK,), jnp.int32),
            pltpu.VMEM((EBLK,), jnp.int32),
            pltpu.VMEM((EBLK,), jnp.int32),
            pltpu.VMEM((EBLK,), jnp.int32),
            pltpu.VMEM((RING, B), jnp.int32),
            pltpu.VMEM((RING, B), jnp.int32),
            pltpu.VMEM((B, D), jnp.float32),
            pltpu.VMEM((B, D), jnp.float32),
            pltpu.SemaphoreType.DMA,
            pltpu.SemaphoreType.DMA,
            pltpu.SemaphoreType.DMA,
            pltpu.SemaphoreType.DMA,
            pltpu.SemaphoreType.DMA,
            pltpu.SemaphoreType.DMA,
        )
    else:
        scratch = (
            pltpu.VMEM_SHARED((ROWSP, D), jnp.float32),
            pltpu.VMEM((EBLK,), jnp.int32),
            pltpu.VMEM((EBLK,), jnp.int32),
            pltpu.VMEM((RING, B), jnp.int32),
            pltpu.VMEM((B, D), jnp.float32),
            pltpu.SemaphoreType.DMA,
            pltpu.SemaphoreType.DMA,
            pltpu.SemaphoreType.DMA,
        )
    return pl.kernel(
        functools.partial(_sc_body, with_gather),
        out_type=(jax.ShapeDtypeStruct((SROWS, D), jnp.float32),),
        mesh=mesh,
        compiler_params=pltpu.CompilerParams(needs_layout_passes=False),
        scratch_types=scratch,
    )


_sc_layer = _make_sc_kernel(True)
_sc_cnt = _make_sc_kernel(False)

BN = 1000


def _tc_body(leaky, x_ref, s_ref, c_ref, root_ref, w_ref, b_ref, g_ref,
             bt_ref, o_ref):
    x = x_ref[...]
    acc = jnp.dot(x, root_ref[...], preferred_element_type=jnp.float32)
    acc = acc + b_ref[...]
    rec = jnp.concatenate(
        [jnp.broadcast_to(jnp.maximum(c_ref[:, r * D:r * D + 1], 1.0),
                          (BN, D)) for r in range(R)], axis=1)
    mean = s_ref[...] / rec
    acc = acc + jnp.dot(mean, w_ref[...], preferred_element_type=jnp.float32)
    if leaky:
        acc = jnp.where(acc > 0, acc, 0.2 * acc)
    mu = jnp.mean(acc, axis=1, keepdims=True)
    var = jnp.mean((acc - mu) ** 2, axis=1, keepdims=True)
    o_ref[...] = (acc - mu) / jnp.sqrt(var + EPS) * g_ref[...] + bt_ref[...]


def _make_tc_kernel(leaky):
    return pl.pallas_call(
        functools.partial(_tc_body, leaky),
        grid=(N // BN,),
        in_specs=[
            pl.BlockSpec((BN, D), lambda i: (i, 0)),
            pl.BlockSpec((BN, R * D), lambda i: (i, 0)),
            pl.BlockSpec((BN, R * D), lambda i: (i, 0)),
            pl.BlockSpec((D, D), lambda i: (0, 0)),
            pl.BlockSpec((R * D, D), lambda i: (0, 0)),
            pl.BlockSpec((1, D), lambda i: (0, 0)),
            pl.BlockSpec((1, D), lambda i: (0, 0)),
            pl.BlockSpec((1, D), lambda i: (0, 0)),
        ],
        out_specs=pl.BlockSpec((BN, D), lambda i: (i, 0)),
        out_shape=jax.ShapeDtypeStruct((N, D), jnp.float32),
    )


_tc_layer1 = _make_tc_kernel(True)
_tc_layer2 = _make_tc_kernel(False)


def kernel(x, edge_index, edge_type, W1, root1, b1, g1, beta1, W2, root2,
           b2, g2, beta2):
    src = edge_index[0]
    dst = edge_index[1]
    key = dst * R + edge_type
    NP = SROWS // R
    (cnt,) = _sc_cnt(key)
    cs = cnt.reshape(NP, R * D)
    (s1,) = _sc_layer(x, key, src)
    h = _tc_layer1(x, s1.reshape(NP, R * D), cs, root1,
                   W1.reshape(R * D, D), b1.reshape(1, D),
                   g1.reshape(1, D), beta1.reshape(1, D))
    (s2,) = _sc_layer(h, key, src)
    out = _tc_layer2(h, s2.reshape(NP, R * D), cs, root2,
                     W2.reshape(R * D, D), b2.reshape(1, D),
                     g2.reshape(1, D), beta2.reshape(1, D))
    return out
